# Initial kernel scaffold; baseline (speedup 1.0000x reference)
import jax
import jax.numpy as jnp
from jax.experimental import pallas as pl


def kernel(xs, gates):
    raise NotImplementedError("write your pallas kernel here")


# speedup vs baseline: 6.7564x; 6.7564x over previous
import jax
import jax.numpy as jnp
import numpy as np
from jax.experimental import pallas as pl
from jax.experimental.pallas import tpu as pltpu

_EPS = float(np.finfo(float).eps)


def _body(x_ref, g_ref, o_ref):
    e_total = x_ref.shape[0]
    acc = jnp.exp(x_ref[0]) * g_ref[:, 0:1]
    for e in range(1, e_total):
        acc = acc + jnp.exp(x_ref[e]) * g_ref[:, e : e + 1]
    o_ref[...] = jnp.log(jnp.where(acc == 0.0, _EPS, acc))


def kernel(xs, gates):
    E, B, P, C = xs.shape
    PC = P * C
    xs_f = xs.reshape(E, B, PC)
    Bb = 32

    out = pl.pallas_call(
        _body,
        grid=(B // Bb,),
        in_specs=[
            pl.BlockSpec((E, Bb, PC), lambda i: (0, i, 0)),
            pl.BlockSpec((Bb, E), lambda i: (i, 0)),
        ],
        out_specs=pl.BlockSpec((Bb, PC), lambda i: (i, 0)),
        out_shape=jax.ShapeDtypeStruct((B, PC), jnp.float32),
    )(xs_f, gates)
    return out.reshape(B, P, C)


# speedup vs baseline: 6.8234x; 1.0099x over previous
import jax
import jax.numpy as jnp
import numpy as np
from jax.experimental import pallas as pl
from jax.experimental.pallas import tpu as pltpu

_EPS = float(np.finfo(float).eps)


def _body(x_ref, g_ref, o_ref):
    e_total = x_ref.shape[0]
    acc = jnp.exp(x_ref[0]) * g_ref[:, 0:1]
    for e in range(1, e_total):
        acc = acc + jnp.exp(x_ref[e]) * g_ref[:, e : e + 1]
    o_ref[...] = jnp.log(jnp.where(acc == 0.0, _EPS, acc))


def kernel(xs, gates):
    E, B, P, C = xs.shape
    PC = P * C
    xs_f = xs.reshape(E, B, PC)
    Bb = 64

    out = pl.pallas_call(
        _body,
        grid=(B // Bb,),
        in_specs=[
            pl.BlockSpec((E, Bb, PC), lambda i: (0, i, 0)),
            pl.BlockSpec((Bb, E), lambda i: (i, 0)),
        ],
        out_specs=pl.BlockSpec((Bb, PC), lambda i: (i, 0)),
        out_shape=jax.ShapeDtypeStruct((B, PC), jnp.float32),
    )(xs_f, gates)
    return out.reshape(B, P, C)
